# baseline (speedup 1.0000x reference)
import jax
import jax.numpy as jnp
from jax import lax
from jax.experimental import pallas as pl
from jax.experimental.pallas import tpu as pltpu
from jax.experimental.pallas import tpu_sc as plsc

VOCAB = 100000
EMBED = 64
SEQ = 200
BATCH = 4096

NC, NS = 2, 16
NW = NC * NS
BT = BATCH // 128
ST = SEQ // 8
SLAB = 2
NSLAB = SEQ // SLAB
ROWS_SLAB = SLAB * 128
BUFP = 129


def _slab_gathers(i, slot, idx_v, tok_hbm, rows_v, gsem):
    st = i >> 2
    r0 = (i & 3) * 2
    descs = []
    for sr in range(SLAB):
        descs.append(pltpu.async_copy(
            tok_hbm.at[idx_v.at[st, r0 + sr]],
            rows_v.at[pl.ds(slot * ROWS_SLAB + sr * 128, 128)],
            gsem,
        ))
    return descs


def _slab_compute(i, slot, rows_v, buf_v, pos_v):
    lanes = lax.iota(jnp.int32, 16)
    e_vec = lanes & 7
    et_off = lanes >> 3
    ps = [[pos_v[2 * i + sr, pl.ds(eb * 16, 16)] for eb in range(4)]
          for sr in range(SLAB)]
    r_vecs = [[et_off + (slot * 16 + sr * 8 + 2 * eb) for eb in range(4)]
              for sr in range(SLAB)]

    def b_body(b, bvec):
        for sr in range(SLAB):
            row = slot * ROWS_SLAB + sr * 128 + b
            for eb in range(4):
                v = rows_v[row, pl.ds(eb * 16, 16)]
                plsc.store_scatter(buf_v, [r_vecs[sr][eb], e_vec, bvec],
                                   v + ps[sr][eb])
        return bvec + 1

    lax.fori_loop(0, 128, b_body, lanes & 0, unroll=4)


def _body(idx_hbm, tok_hbm, pos_hbm, out_hbm, idx_v, pos_v, rows_v, buf_v,
          gsem0, gsem1, osem0, osem1):
    wid = lax.axis_index("s") * NC + lax.axis_index("c")
    gsems = (gsem0, gsem1)
    osems = (osem0, osem1)

    pltpu.sync_copy(idx_hbm.at[:, wid], idx_v)
    pltpu.sync_copy(pos_hbm, pos_v)

    _slab_gathers(0, 0, idx_v, tok_hbm, rows_v, gsems[0])

    def out_copy(i, slot, osem):
        return pltpu.async_copy(
            buf_v.at[pl.ds(slot * 16, 16), :, pl.ds(0, 128)],
            out_hbm.at[pl.ds(16 * i, 16), wid],
            osem,
        )

    def iteration(i, _):
        for slot in range(2):
            @pl.when((i & 1) == slot)
            def _():
                other = 1 - slot
                @pl.when(i + 1 < NSLAB)
                def _():
                    _slab_gathers(i + 1, other, idx_v, tok_hbm, rows_v,
                                  gsems[other])
                for sr in range(SLAB):
                    pltpu.make_async_copy(
                        tok_hbm.at[idx_v.at[0, 0]],
                        rows_v.at[pl.ds(slot * ROWS_SLAB + sr * 128, 128)],
                        gsems[slot],
                    ).wait()
                @pl.when(i >= 2)
                def _():
                    pltpu.make_async_copy(
                        buf_v.at[pl.ds(slot * 16, 16), :, pl.ds(0, 128)],
                        out_hbm.at[pl.ds(16 * (i - 2), 16), wid],
                        osems[slot],
                    ).wait()
                _slab_compute(i, slot, rows_v, buf_v, pos_v)
                out_copy(i, slot, osems[slot])
        return ()

    lax.fori_loop(0, NSLAB, iteration, ())

    for i in (NSLAB - 2, NSLAB - 1):
        slot = i % 2
        pltpu.make_async_copy(
            buf_v.at[pl.ds(slot * 16, 16), :, pl.ds(0, 128)],
            out_hbm.at[pl.ds(16 * i, 16), wid],
            osems[slot],
        ).wait()


@jax.jit
def _embed(idx4d, token_table, pos_table):
    mesh = plsc.VectorSubcoreMesh(core_axis_name="c", subcore_axis_name="s")
    return pl.kernel(
        _body,
        out_type=jax.ShapeDtypeStruct((SEQ * 8, BT, 8, 128), jnp.float32),
        mesh=mesh,
        compiler_params=pltpu.CompilerParams(
            needs_layout_passes=False, use_tc_tiling_on_sc=False),
        scratch_types=[
            pltpu.VMEM((ST, 8, 128), jnp.int32),
            pltpu.VMEM((SEQ, EMBED), jnp.float32),
            pltpu.VMEM((2 * ROWS_SLAB, EMBED), jnp.float32),
            pltpu.VMEM((32, 8, BUFP), jnp.float32),
            pltpu.SemaphoreType.DMA,
            pltpu.SemaphoreType.DMA,
            pltpu.SemaphoreType.DMA,
            pltpu.SemaphoreType.DMA,
        ],
    )(idx4d, token_table, pos_table)


def kernel(inputs, token_table, pos_table):
    idx4d = (inputs.astype(jnp.int32)
             .reshape(BT, 128, ST, 8).transpose(2, 0, 3, 1))
    out = _embed(idx4d, token_table, pos_table)
    return (out.reshape(SEQ, 8, BT, 8, 128)
            .transpose(2, 4, 0, 1, 3).reshape(BATCH, SEQ, EMBED))


# speedup vs baseline: 2.5339x; 2.5339x over previous
import jax
import jax.numpy as jnp
from jax import lax
from jax.experimental import pallas as pl
from jax.experimental.pallas import tpu as pltpu
from jax.experimental.pallas import tpu_sc as plsc

VOCAB = 100000
EMBED = 64
SEQ = 200
BATCH = 4096

NC, NS = 2, 16
NW = NC * NS
BT = BATCH // 128
ST = SEQ // 8
SLAB = 2
NSLAB = SEQ // SLAB
ROWS_SLAB = SLAB * 128
BUFP = 129


def _slab_gathers(i, slot, idx_v, tok_hbm, rows_v, gsem):
    st = i >> 2
    r0 = (i & 3) * 2
    descs = []
    for sr in range(SLAB):
        descs.append(pltpu.async_copy(
            tok_hbm.at[idx_v.at[st, r0 + sr]],
            rows_v.at[pl.ds(slot * ROWS_SLAB + sr * 128, 128)],
            gsem,
        ))
    return descs


def _slab_compute(i, slot, rows_v, buf_v, pos_v):
    lanes = lax.iota(jnp.int32, 16)
    e_vec = lanes & 7
    et_off = lanes >> 3
    ps = [[pos_v[2 * i + sr, pl.ds(eb * 16, 16)] for eb in range(4)]
          for sr in range(SLAB)]
    r_vecs = [[et_off + (slot * 16 + sr * 8 + 2 * eb) for eb in range(4)]
              for sr in range(SLAB)]

    @plsc.parallel_loop(0, 128, unroll=4, carry=lanes & 0)
    def b_body(b, bvec):
        for sr in range(SLAB):
            row = slot * ROWS_SLAB + sr * 128 + b
            for eb in range(4):
                v = rows_v[row, pl.ds(eb * 16, 16)]
                plsc.store_scatter(buf_v, [r_vecs[sr][eb], e_vec, bvec],
                                   v + ps[sr][eb])
        return bvec + 1


def _body(idx_hbm, tok_hbm, pos_hbm, out_hbm, idx_v, pos_v, rows_v, buf_v,
          gsem0, gsem1, osem0, osem1):
    wid = lax.axis_index("s") * NC + lax.axis_index("c")
    gsems = (gsem0, gsem1)
    osems = (osem0, osem1)

    pltpu.sync_copy(idx_hbm.at[:, wid], idx_v)
    pltpu.sync_copy(pos_hbm, pos_v)

    _slab_gathers(0, 0, idx_v, tok_hbm, rows_v, gsems[0])

    def out_copy(i, slot, osem):
        return pltpu.async_copy(
            buf_v.at[pl.ds(slot * 16, 16), :, pl.ds(0, 128)],
            out_hbm.at[pl.ds(16 * i, 16), wid],
            osem,
        )

    def iteration(i, _):
        for slot in range(2):
            @pl.when((i & 1) == slot)
            def _():
                other = 1 - slot
                @pl.when(i + 1 < NSLAB)
                def _():
                    _slab_gathers(i + 1, other, idx_v, tok_hbm, rows_v,
                                  gsems[other])
                for sr in range(SLAB):
                    pltpu.make_async_copy(
                        tok_hbm.at[idx_v.at[0, 0]],
                        rows_v.at[pl.ds(slot * ROWS_SLAB + sr * 128, 128)],
                        gsems[slot],
                    ).wait()
                @pl.when(i >= 2)
                def _():
                    pltpu.make_async_copy(
                        buf_v.at[pl.ds(slot * 16, 16), :, pl.ds(0, 128)],
                        out_hbm.at[pl.ds(16 * (i - 2), 16), wid],
                        osems[slot],
                    ).wait()
                _slab_compute(i, slot, rows_v, buf_v, pos_v)
                out_copy(i, slot, osems[slot])
        return ()

    lax.fori_loop(0, NSLAB, iteration, ())

    for i in (NSLAB - 2, NSLAB - 1):
        slot = i % 2
        pltpu.make_async_copy(
            buf_v.at[pl.ds(slot * 16, 16), :, pl.ds(0, 128)],
            out_hbm.at[pl.ds(16 * i, 16), wid],
            osems[slot],
        ).wait()


@jax.jit
def _embed(idx4d, token_table, pos_table):
    mesh = plsc.VectorSubcoreMesh(core_axis_name="c", subcore_axis_name="s")
    return pl.kernel(
        _body,
        out_type=jax.ShapeDtypeStruct((SEQ * 8, BT, 8, 128), jnp.float32),
        mesh=mesh,
        compiler_params=pltpu.CompilerParams(
            needs_layout_passes=False, use_tc_tiling_on_sc=False),
        scratch_types=[
            pltpu.VMEM((ST, 8, 128), jnp.int32),
            pltpu.VMEM((SEQ, EMBED), jnp.float32),
            pltpu.VMEM((2 * ROWS_SLAB, EMBED), jnp.float32),
            pltpu.VMEM((32, 8, BUFP), jnp.float32),
            pltpu.SemaphoreType.DMA,
            pltpu.SemaphoreType.DMA,
            pltpu.SemaphoreType.DMA,
            pltpu.SemaphoreType.DMA,
        ],
    )(idx4d, token_table, pos_table)


def kernel(inputs, token_table, pos_table):
    idx4d = (inputs.astype(jnp.int32)
             .reshape(BT, 128, ST, 8).transpose(2, 0, 3, 1))
    out = _embed(idx4d, token_table, pos_table)
    return (out.reshape(SEQ, 8, BT, 8, 128)
            .transpose(2, 4, 0, 1, 3).reshape(BATCH, SEQ, EMBED))


# speedup vs baseline: 2.6720x; 1.0545x over previous
import jax
import jax.numpy as jnp
from jax import lax
from jax.experimental import pallas as pl
from jax.experimental.pallas import tpu as pltpu
from jax.experimental.pallas import tpu_sc as plsc

VOCAB = 100000
EMBED = 64
SEQ = 200
BATCH = 4096

NC, NS = 2, 16
NW = NC * NS
BT = BATCH // 128
ST = SEQ // 8
SLAB = 2
NSLAB = SEQ // SLAB
ROWS_SLAB = SLAB * 128
BUFP = 129


def _slab_gathers(i, slot, idx_v, tok_hbm, rows_v, gsem):
    st = i >> 2
    r0 = (i & 3) * 2
    descs = []
    for sr in range(SLAB):
        descs.append(pltpu.async_copy(
            tok_hbm.at[idx_v.at[st, r0 + sr]],
            rows_v.at[pl.ds(slot * ROWS_SLAB + sr * 128, 128)],
            gsem,
        ))
    return descs


def _slab_compute(i, rbase, bslot16, rows_v, buf_v, pos_v):
    lanes = lax.iota(jnp.int32, 16)
    e_vec = lanes & 7
    et_off16 = (lanes >> 3) + bslot16
    ps = [[pos_v[2 * i + sr, pl.ds(eb * 16, 16)] for eb in range(4)]
          for sr in range(SLAB)]
    r_vecs = [[et_off16 + (sr * 8 + 2 * eb) for eb in range(4)]
              for sr in range(SLAB)]

    @plsc.parallel_loop(0, 128, unroll=4, carry=lanes & 0)
    def b_body(b, bvec):
        for sr in range(SLAB):
            row = rbase + sr * 128 + b
            for eb in range(4):
                v = rows_v[row, pl.ds(eb * 16, 16)]
                plsc.store_scatter(buf_v, [r_vecs[sr][eb], e_vec, bvec],
                                   v + ps[sr][eb])
        return bvec + 1


def _body(idx_hbm, tok_hbm, pos_hbm, out_hbm, idx_v, pos_v, rows_v, buf_v,
          gsem0, gsem1, gsem2, osem0, osem1):
    wid = lax.axis_index("s") * NC + lax.axis_index("c")
    gsems = (gsem0, gsem1, gsem2)
    osems = (osem0, osem1)

    pltpu.sync_copy(idx_hbm.at[:, wid], idx_v)
    pltpu.sync_copy(pos_hbm, pos_v)

    _slab_gathers(0, 0, idx_v, tok_hbm, rows_v, gsems[0])
    _slab_gathers(1, 1, idx_v, tok_hbm, rows_v, gsems[1])

    def out_copy(i, slot, osem):
        return pltpu.async_copy(
            buf_v.at[pl.ds(slot * 16, 16), :, pl.ds(0, 128)],
            out_hbm.at[pl.ds(16 * i, 16), wid],
            osem,
        )

    def iteration(i, _):
        rslot = lax.rem(i, 3)
        for rs in range(3):
            @pl.when(jnp.logical_and(lax.rem(i + 2, 3) == rs, i + 2 < NSLAB))
            def _():
                _slab_gathers(i + 2, rs, idx_v, tok_hbm, rows_v, gsems[rs])
        for rs in range(3):
            @pl.when(rslot == rs)
            def _():
                for sr in range(SLAB):
                    pltpu.make_async_copy(
                        tok_hbm.at[idx_v.at[0, 0]],
                        rows_v.at[pl.ds(rs * ROWS_SLAB + sr * 128, 128)],
                        gsems[rs],
                    ).wait()
        for bs in range(2):
            @pl.when(jnp.logical_and((i & 1) == bs, i >= 2))
            def _():
                pltpu.make_async_copy(
                    buf_v.at[pl.ds(bs * 16, 16), :, pl.ds(0, 128)],
                    out_hbm.at[pl.ds(16 * (i - 2), 16), wid],
                    osems[bs],
                ).wait()
        _slab_compute(i, rslot * ROWS_SLAB, (i & 1) * 16, rows_v, buf_v,
                      pos_v)
        for bs in range(2):
            @pl.when((i & 1) == bs)
            def _():
                out_copy(i, bs, osems[bs])
        return ()

    lax.fori_loop(0, NSLAB, iteration, ())

    for i in (NSLAB - 2, NSLAB - 1):
        slot = i % 2
        pltpu.make_async_copy(
            buf_v.at[pl.ds(slot * 16, 16), :, pl.ds(0, 128)],
            out_hbm.at[pl.ds(16 * i, 16), wid],
            osems[slot],
        ).wait()


@jax.jit
def _embed(idx4d, token_table, pos_table):
    mesh = plsc.VectorSubcoreMesh(core_axis_name="c", subcore_axis_name="s")
    return pl.kernel(
        _body,
        out_type=jax.ShapeDtypeStruct((SEQ * 8, BT, 8, 128), jnp.float32),
        mesh=mesh,
        compiler_params=pltpu.CompilerParams(
            needs_layout_passes=False, use_tc_tiling_on_sc=False),
        scratch_types=[
            pltpu.VMEM((ST, 8, 128), jnp.int32),
            pltpu.VMEM((SEQ, EMBED), jnp.float32),
            pltpu.VMEM((3 * ROWS_SLAB, EMBED), jnp.float32),
            pltpu.VMEM((32, 8, BUFP), jnp.float32),
            pltpu.SemaphoreType.DMA,
            pltpu.SemaphoreType.DMA,
            pltpu.SemaphoreType.DMA,
            pltpu.SemaphoreType.DMA,
            pltpu.SemaphoreType.DMA,
        ],
    )(idx4d, token_table, pos_table)


def kernel(inputs, token_table, pos_table):
    idx4d = (inputs.astype(jnp.int32)
             .reshape(BT, 128, ST, 8).transpose(2, 0, 3, 1))
    out = _embed(idx4d, token_table, pos_table)
    return (out.reshape(SEQ, 8, BT, 8, 128)
            .transpose(2, 4, 0, 1, 3).reshape(BATCH, SEQ, EMBED))
